# Initial kernel scaffold; baseline (speedup 1.0000x reference)
import jax
import jax.numpy as jnp
from jax.experimental import pallas as pl


def kernel(src_xyz, src_desc, src_scores, dst_xyz, dst_desc, dst_scores, epoch):
    raise NotImplementedError("write your pallas kernel here")


# speedup vs baseline: 1.1539x; 1.1539x over previous
import functools

import numpy as np
import jax
import jax.numpy as jnp
from jax import lax
from jax.experimental import pallas as pl
from jax.experimental.pallas import tpu as pltpu

POS_RADIUS = 0.1
NEG_RADIUS = 0.2
TRIPLET_MARGIN = 1.0
MAX_ANCHOR = 1024
MAX_DST = 8192
VOTING_START = 0

_N = 20000
_rng = np.random.RandomState(0)
_PERM_SRC = jnp.asarray(_rng.permutation(_N)[:MAX_ANCHOR], dtype=jnp.int32)
_PERM_DST = jnp.asarray(_rng.permutation(_N)[:MAX_DST], dtype=jnp.int32)

_NN_BLK = 2048
_N_PAD = ((_N + _NN_BLK - 1) // _NN_BLK) * _NN_BLK

_NEG_BLK = 2048


def _nn_kernel(a_ref, sa_ref, b_ref, mind_ref, idx_ref):
    j = pl.program_id(0)
    a = a_ref[...]
    b = b_ref[...]
    sa = sa_ref[...]
    sb = jnp.sum(b * b, axis=1)
    prod = lax.dot_general(a, b, (((1,), (1,)), ((), ())),
                           preferred_element_type=jnp.float32)
    d2 = sa[:, None] + sb[None, :] - 2.0 * prod
    d = jnp.sqrt(jnp.maximum(d2, 1e-12))
    col = j * _NN_BLK + lax.broadcasted_iota(jnp.int32, d.shape, 1)
    d = jnp.where(col < _N, d, jnp.inf)
    blk_min = jnp.min(d, axis=1)
    blk_idx = jnp.min(jnp.where(d == blk_min[:, None], col, _N), axis=1)

    @pl.when(j == 0)
    def _():
        mind_ref[...] = blk_min
        idx_ref[...] = blk_idx

    @pl.when(j > 0)
    def _():
        prev = mind_ref[...]
        better = blk_min < prev
        mind_ref[...] = jnp.where(better, blk_min, prev)
        idx_ref[...] = jnp.where(better, blk_idx, idx_ref[...])


def _nn_search(pc_src, dst_xyz_pad):
    sa = jnp.sum(pc_src * pc_src, axis=1)
    grid = _N_PAD // _NN_BLK
    mind, idx = pl.pallas_call(
        _nn_kernel,
        grid=(grid,),
        in_specs=[
            pl.BlockSpec((MAX_ANCHOR, 4), lambda j: (0, 0)),
            pl.BlockSpec((MAX_ANCHOR,), lambda j: (0,)),
            pl.BlockSpec((_NN_BLK, 4), lambda j: (j, 0)),
        ],
        out_specs=[
            pl.BlockSpec((MAX_ANCHOR,), lambda j: (0,)),
            pl.BlockSpec((MAX_ANCHOR,), lambda j: (0,)),
        ],
        out_shape=[
            jax.ShapeDtypeStruct((MAX_ANCHOR,), jnp.float32),
            jax.ShapeDtypeStruct((MAX_ANCHOR,), jnp.int32),
        ],
    )(pc_src, sa, dst_xyz_pad)
    return mind, idx


def _loss_kernel(a_ref, sa_ref, ad_ref, sad_ref, b_ref, bd_ref,
                 pos_ref, sig_ref, nnd_ref, out_ref, negmin_ref):
    j = pl.program_id(0)
    nblk = pl.num_programs(0)
    a = a_ref[...]
    b = b_ref[...]
    ad = ad_ref[...]
    bd = bd_ref[...]
    sa = sa_ref[...]
    sad = sad_ref[...]
    sb = jnp.sum(b * b, axis=1)
    sbd = jnp.sum(bd * bd, axis=1)

    prod_x = lax.dot_general(a, b, (((1,), (1,)), ((), ())),
                             preferred_element_type=jnp.float32)
    dist2 = sa[:, None] + sb[None, :] - 2.0 * prod_x
    dist = jnp.sqrt(jnp.maximum(dist2, 1e-12))

    prod_d = lax.dot_general(ad, bd, (((1,), (1,)), ((), ())),
                             preferred_element_type=jnp.float32)
    desc2 = sad[:, None] + sbd[None, :] - 2.0 * prod_d
    desc = jnp.sqrt(jnp.maximum(desc2, 1e-12))
    desc = desc + jnp.where(dist < NEG_RADIUS, 1e10, 0.0)
    blk_min = jnp.min(desc, axis=1)

    @pl.when(j == 0)
    def _():
        negmin_ref[...] = blk_min

    @pl.when(j > 0)
    def _():
        negmin_ref[...] = jnp.minimum(negmin_ref[...], blk_min)

    @pl.when(j == nblk - 1)
    def _():
        negative_min = negmin_ref[...]
        pos = pos_ref[...]
        diff = ad - pos
        positive_max = jnp.sqrt(jnp.sum(diff * diff, axis=1) + 1e-12)
        p_n_diff = positive_max - negative_min
        nnd = nnd_ref[...]
        maskf = (nnd < POS_RADIUS).astype(jnp.float32)
        count = jnp.sum(maskf)
        desc_loss = jnp.sum(jnp.maximum(p_n_diff + TRIPLET_MARGIN, 0.0) * maskf)
        score_loss = jnp.sum(sig_ref[...] * p_n_diff * maskf)
        loss = (desc_loss + score_loss) / count
        loss = jnp.where(count < float(MAX_ANCHOR // 2), 0.0, loss)
        out_ref[...] = loss.reshape(1, 1)


def _main_loss(pc_src, anc_desc, pc_dst_sub, desc_dst_sub, pos_desc,
               sel_sigma, nn_d):
    sa = jnp.sum(pc_src * pc_src, axis=1)
    sad = jnp.sum(anc_desc * anc_desc, axis=1)
    grid = MAX_DST // _NEG_BLK
    out = pl.pallas_call(
        _loss_kernel,
        grid=(grid,),
        in_specs=[
            pl.BlockSpec((MAX_ANCHOR, 4), lambda j: (0, 0)),
            pl.BlockSpec((MAX_ANCHOR,), lambda j: (0,)),
            pl.BlockSpec((MAX_ANCHOR, 64), lambda j: (0, 0)),
            pl.BlockSpec((MAX_ANCHOR,), lambda j: (0,)),
            pl.BlockSpec((_NEG_BLK, 4), lambda j: (j, 0)),
            pl.BlockSpec((_NEG_BLK, 64), lambda j: (j, 0)),
            pl.BlockSpec((MAX_ANCHOR, 64), lambda j: (0, 0)),
            pl.BlockSpec((MAX_ANCHOR,), lambda j: (0,)),
            pl.BlockSpec((MAX_ANCHOR,), lambda j: (0,)),
        ],
        out_specs=pl.BlockSpec((1, 1), lambda j: (0, 0)),
        out_shape=jax.ShapeDtypeStruct((1, 1), jnp.float32),
        scratch_shapes=[pltpu.VMEM((MAX_ANCHOR,), jnp.float32)],
    )(pc_src, sa, anc_desc, sad, pc_dst_sub, desc_dst_sub,
      pos_desc, sel_sigma, nn_d)
    return out[0, 0]


def kernel(src_xyz, src_desc, src_scores, dst_xyz, dst_desc, dst_scores, epoch):
    pc_src = jnp.pad(src_xyz[_PERM_SRC], ((0, 0), (0, 1)))
    anc_desc = src_desc[_PERM_SRC]
    dst_xyz_pad = jnp.pad(dst_xyz, ((0, _N_PAD - _N), (0, 1)))
    nn_d, nn = _nn_search(pc_src, dst_xyz_pad)

    pos_desc = dst_desc[nn]
    sel_sigma = (src_scores[_PERM_SRC] + dst_scores[nn]) * 0.5
    pc_dst_sub = jnp.pad(dst_xyz[_PERM_DST], ((0, 0), (0, 1)))
    desc_dst_sub = dst_desc[_PERM_DST]

    loss = _main_loss(pc_src, anc_desc, pc_dst_sub, desc_dst_sub,
                      pos_desc, sel_sigma, nn_d)
    out = jnp.where(jnp.asarray(epoch) <= VOTING_START, 0.0, loss)
    return out.astype(jnp.float32)


# speedup vs baseline: 1.2369x; 1.0720x over previous
import functools

import numpy as np
import jax
import jax.numpy as jnp
from jax import lax
from jax.experimental import pallas as pl
from jax.experimental.pallas import tpu as pltpu

POS_RADIUS = 0.1
NEG_RADIUS = 0.2
TRIPLET_MARGIN = 1.0
MAX_ANCHOR = 1024
MAX_DST = 8192
VOTING_START = 0

_N = 20000
_rng = np.random.RandomState(0)
_PERM_SRC = np.ascontiguousarray(_rng.permutation(_N)[:MAX_ANCHOR].astype(np.int32))
_PERM_DST = np.ascontiguousarray(_rng.permutation(_N)[:MAX_DST].astype(np.int32))

_NN_BLK = 2000
_NEG_BLK = 2048


def _nn_kernel(a_ref, b_ref, mind_ref, idx_ref):
    j = pl.program_id(0)
    a = a_ref[...]
    b = b_ref[...]
    sa = jnp.sum(a * a, axis=1)
    sb = jnp.sum(b * b, axis=1)
    prod = lax.dot_general(a, b, (((1,), (1,)), ((), ())),
                           preferred_element_type=jnp.float32)
    d2 = (sa[:, None] + sb[None, :]) - 2.0 * prod
    d = jnp.sqrt(jnp.maximum(d2, 1e-12))
    col = j * _NN_BLK + lax.broadcasted_iota(jnp.int32, d.shape, 1)
    blk_min = jnp.min(d, axis=1)
    blk_idx = jnp.min(jnp.where(d == blk_min[:, None], col, _N), axis=1)

    @pl.when(j == 0)
    def _():
        mind_ref[...] = blk_min
        idx_ref[...] = blk_idx

    @pl.when(j > 0)
    def _():
        prev = mind_ref[...]
        better = blk_min < prev
        mind_ref[...] = jnp.where(better, blk_min, prev)
        idx_ref[...] = jnp.where(better, blk_idx, idx_ref[...])


def _nn_search(pc_src, dst_xyz):
    grid = _N // _NN_BLK
    mind, idx = pl.pallas_call(
        _nn_kernel,
        grid=(grid,),
        in_specs=[
            pl.BlockSpec((MAX_ANCHOR, 3), lambda j: (0, 0)),
            pl.BlockSpec((_NN_BLK, 3), lambda j: (j, 0)),
        ],
        out_specs=[
            pl.BlockSpec((MAX_ANCHOR,), lambda j: (0,)),
            pl.BlockSpec((MAX_ANCHOR,), lambda j: (0,)),
        ],
        out_shape=[
            jax.ShapeDtypeStruct((MAX_ANCHOR,), jnp.float32),
            jax.ShapeDtypeStruct((MAX_ANCHOR,), jnp.int32),
        ],
    )(pc_src, dst_xyz)
    return mind, idx


def _loss_kernel(a_ref, ad_ref, b_ref, bd_ref,
                 pos_ref, sig_ref, nnd_ref, out_ref, negmin_ref):
    j = pl.program_id(0)
    nblk = pl.num_programs(0)
    a = a_ref[...]
    b = b_ref[...]
    ad = ad_ref[...]
    bd = bd_ref[...]
    sa = jnp.sum(a * a, axis=1)
    sad = jnp.sum(ad * ad, axis=1)
    sb = jnp.sum(b * b, axis=1)
    sbd = jnp.sum(bd * bd, axis=1)

    prod_x = lax.dot_general(a, b, (((1,), (1,)), ((), ())),
                             preferred_element_type=jnp.float32)
    dist2 = (sa[:, None] + sb[None, :]) - 2.0 * prod_x
    dist = jnp.sqrt(jnp.maximum(dist2, 1e-12))

    prod_d = lax.dot_general(ad, bd, (((1,), (1,)), ((), ())),
                             preferred_element_type=jnp.float32)
    desc2 = (sad[:, None] + sbd[None, :]) - 2.0 * prod_d
    desc = jnp.sqrt(jnp.maximum(desc2, 1e-12))
    desc = desc + jnp.where(dist < NEG_RADIUS, 1e10, 0.0)
    blk_min = jnp.min(desc, axis=1)

    @pl.when(j == 0)
    def _():
        negmin_ref[...] = blk_min

    @pl.when(j > 0)
    def _():
        negmin_ref[...] = jnp.minimum(negmin_ref[...], blk_min)

    @pl.when(j == nblk - 1)
    def _():
        negative_min = negmin_ref[...]
        pos = pos_ref[...]
        diff = ad - pos
        positive_max = jnp.sqrt(jnp.sum(diff * diff, axis=1) + 1e-12)
        p_n_diff = positive_max - negative_min
        nnd = nnd_ref[...]
        maskf = (nnd < POS_RADIUS).astype(jnp.float32)
        count = jnp.sum(maskf)
        desc_loss = jnp.sum(jnp.maximum(p_n_diff + TRIPLET_MARGIN, 0.0) * maskf)
        score_loss = jnp.sum(sig_ref[...] * p_n_diff * maskf)
        loss = (desc_loss + score_loss) / count
        loss = jnp.where(count < float(MAX_ANCHOR // 2), 0.0, loss)
        out_ref[...] = loss.reshape(1, 1)


def _main_loss(pc_src, anc_desc, pc_dst_sub, desc_dst_sub, pos_desc,
               sel_sigma, nn_d):
    grid = MAX_DST // _NEG_BLK
    out = pl.pallas_call(
        _loss_kernel,
        grid=(grid,),
        in_specs=[
            pl.BlockSpec((MAX_ANCHOR, 3), lambda j: (0, 0)),
            pl.BlockSpec((MAX_ANCHOR, 64), lambda j: (0, 0)),
            pl.BlockSpec((_NEG_BLK, 3), lambda j: (j, 0)),
            pl.BlockSpec((_NEG_BLK, 64), lambda j: (j, 0)),
            pl.BlockSpec((MAX_ANCHOR, 64), lambda j: (0, 0)),
            pl.BlockSpec((MAX_ANCHOR,), lambda j: (0,)),
            pl.BlockSpec((MAX_ANCHOR,), lambda j: (0,)),
        ],
        out_specs=pl.BlockSpec((1, 1), lambda j: (0, 0)),
        out_shape=jax.ShapeDtypeStruct((1, 1), jnp.float32),
        scratch_shapes=[pltpu.VMEM((MAX_ANCHOR,), jnp.float32)],
    )(pc_src, anc_desc, pc_dst_sub, desc_dst_sub,
      pos_desc, sel_sigma, nn_d)
    return out[0, 0]


def kernel(src_xyz, src_desc, src_scores, dst_xyz, dst_desc, dst_scores, epoch):
    pc_src = src_xyz[_PERM_SRC]
    anc_desc = src_desc[_PERM_SRC]
    nn_d, nn = _nn_search(pc_src, dst_xyz)

    pos_desc = dst_desc[nn]
    sel_sigma = (src_scores[_PERM_SRC] + dst_scores[nn]) * 0.5
    pc_dst_sub = dst_xyz[_PERM_DST]
    desc_dst_sub = dst_desc[_PERM_DST]

    loss = _main_loss(pc_src, anc_desc, pc_dst_sub, desc_dst_sub,
                      pos_desc, sel_sigma, nn_d)
    out = jnp.where(jnp.asarray(epoch) <= VOTING_START, 0.0, loss)
    return out.astype(jnp.float32)
